# Initial kernel scaffold; baseline (speedup 1.0000x reference)
import jax
import jax.numpy as jnp
from jax.experimental import pallas as pl


def kernel(values, sample_ids):
    raise NotImplementedError("write your pallas kernel here")


# speedup vs baseline: 1.6562x; 1.6562x over previous
import jax
import jax.numpy as jnp
from jax import lax
from jax.experimental import pallas as pl
from jax.experimental.pallas import tpu as pltpu
from jax.experimental.pallas import tpu_sc as plsc

N = 32768
D = 128
NC = 2
NS = 16
L = 16
CH = N // NS
GR = N // (NC * NS)
NB1 = 256
NB2 = 128


def _body(values_hbm, keys_hbm, out_hbm,
          ck, posb, hist, gbuf, cnt, idxb, rowb0, rowb1,
          gs_s, a_s, ord_s,
          sem_sc, sem_g0, sem_g1):
  s = lax.axis_index("s")
  c = lax.axis_index("c")

  probe, _ = plsc.scan_count(jnp.zeros((L,), jnp.int32))
  bias = jnp.min(probe)
  one_m_bias = 1 - bias

  wlt = [(jnp.int32(w) < s).astype(jnp.int32) for w in range(NS)]

  pltpu.sync_copy(keys_hbm.at[pl.ds(s * CH, CH)], ck)

  @pl.loop(jnp.int32(0), jnp.int32(CH // L))
  def _build(i):
    sl = pl.ds(i * L, L)
    idx = s * CH + i * L + lax.iota(jnp.int32, L)
    ck[sl] = ck[sl] * 32768 + idx

  def counting_pass(shift, nbins, dst_ref, final):
    nvb = nbins // L

    @pl.loop(jnp.int32(0), jnp.int32(nvb))
    def _zero(b):
      hist[pl.ds(b * L, L)] = jnp.zeros((L,), jnp.int32)

    @pl.loop(jnp.int32(0), jnp.int32(CH // L))
    def _hist(i):
      d = lax.shift_right_logical(ck[pl.ds(i * L, L)], jnp.full((L,), shift, jnp.int32)) & (nbins - 1)
      run, last = plsc.scan_count(d)
      plsc.addupdate_scatter(hist, [d], run + one_m_bias, mask=last)

    pltpu.sync_copy(hist.at[pl.ds(0, nbins)], gs_s.at[s, pl.ds(0, nbins)])
    plsc.subcore_barrier()
    pltpu.sync_copy(gs_s, gbuf)

    @pl.loop(jnp.int32(0), jnp.int32(nvb))
    def _sums(b):
      sl = pl.ds(b * L, L)
      tot = jnp.zeros((L,), jnp.int32)
      part = jnp.zeros((L,), jnp.int32)
      for w in range(NS):
        v = gbuf[w, sl]
        tot = tot + v
        part = part + v * wlt[w]
      hist[sl] = tot
      cnt[sl] = part

    @pl.loop(jnp.int32(0), jnp.int32(nvb), init_carry=jnp.int32(0))
    def _scan(b, carry):
      sl = pl.ds(b * L, L)
      tot = hist[sl]
      cnt[sl] = cnt[sl] + plsc.cumsum(tot) - tot + carry
      return carry + jnp.sum(tot, dtype=jnp.int32)

    @pl.loop(jnp.int32(0), jnp.int32(CH // L))
    def _rank(i):
      sl = pl.ds(i * L, L)
      cvec = ck[sl]
      d = lax.shift_right_logical(cvec, jnp.full((L,), shift, jnp.int32)) & (nbins - 1)
      run, last = plsc.scan_count(d)
      cur = plsc.load_gather(cnt, [d])
      ti = lax.div(i, jnp.int32(8))
      ci = lax.rem(i, jnp.int32(8)) * L
      posb[ti, pl.ds(ci, L)] = cur + run - bias
      plsc.addupdate_scatter(cnt, [d], run + one_m_bias, mask=last)

    if final:
      @pl.loop(jnp.int32(0), jnp.int32(CH // L))
      def _payload(i):
        sl = pl.ds(i * L, L)
        ck[sl] = ck[sl] & 32767

    descs = []
    for t in range(CH // 128):
      descs.append(pltpu.async_copy(
          ck.at[pl.ds(t * 128, 128)], dst_ref.at[posb.at[jnp.int32(t)]], sem_sc))
    for dsc in descs:
      dsc.wait()
    plsc.subcore_barrier()

  counting_pass(15, NB1, a_s, final=False)
  pltpu.sync_copy(a_s.at[pl.ds(s * CH, CH)], ck)
  counting_pass(23, NB2, ord_s, final=True)

  gbase = (s * NC + c) * GR
  for r in range(GR // 128):
    pltpu.sync_copy(ord_s.at[pl.ds(gbase + r * 128, 128)], idxb.at[jnp.int32(r)])

  bufs = (rowb0, rowb1)
  sems = (sem_g0, sem_g1)
  descs = [None, None]
  descs[0] = pltpu.async_copy(values_hbm.at[idxb.at[jnp.int32(0)]], bufs[0], sems[0])
  for r in range(GR // 128):
    if r + 1 < GR // 128:
      descs[(r + 1) % 2] = pltpu.async_copy(
          values_hbm.at[idxb.at[jnp.int32(r + 1)]], bufs[(r + 1) % 2], sems[(r + 1) % 2])
    descs[r % 2].wait()
    pltpu.sync_copy(bufs[r % 2], out_hbm.at[pl.ds(gbase + r * 128, 128)])


@jax.jit
def kernel(values, sample_ids):
  keys32 = sample_ids.astype(jnp.int32)
  mesh = plsc.VectorSubcoreMesh(
      core_axis_name="c", subcore_axis_name="s",
      num_cores=NC, num_subcores=NS)
  fn = pl.kernel(
      _body,
      out_type=jax.ShapeDtypeStruct((N, D), jnp.float32),
      mesh=mesh,
      scratch_types=[
          pltpu.VMEM((CH,), jnp.int32),
          pltpu.VMEM((CH // 128, 128), jnp.int32),
          pltpu.VMEM((NB1,), jnp.int32),
          pltpu.VMEM((NS, NB1), jnp.int32),
          pltpu.VMEM((NB1,), jnp.int32),
          pltpu.VMEM((GR // 128, 128), jnp.int32),
          pltpu.VMEM((128, D), jnp.float32),
          pltpu.VMEM((128, D), jnp.float32),
          pltpu.VMEM_SHARED((NS, NB1), jnp.int32),
          pltpu.VMEM_SHARED((N,), jnp.int32),
          pltpu.VMEM_SHARED((N,), jnp.int32),
          pltpu.SemaphoreType.DMA,
          pltpu.SemaphoreType.DMA,
          pltpu.SemaphoreType.DMA,
      ],
      compiler_params=pltpu.CompilerParams(needs_layout_passes=False),
      name="densify_sc",
  )
  return fn(values, keys32)


# speedup vs baseline: 1.7199x; 1.0384x over previous
import jax
import jax.numpy as jnp
from jax import lax
from jax.experimental import pallas as pl
from jax.experimental.pallas import tpu as pltpu
from jax.experimental.pallas import tpu_sc as plsc

N = 32768
D = 128
NC = 2
NS = 16
L = 16
CH = N // NS
GR = N // (NC * NS)
NB1 = 256
NB2 = 128
NBLK = CH // 128


def _body(values_hbm, keys_hbm, out_hbm,
          ck, dbuf, plb, posb, cnt, gbuf, idxb, rowb0, rowb1,
          gs_s, a_s, ord_s,
          sem_sc, sem_g0, sem_g1, sem_w):
  s = lax.axis_index("s")
  c = lax.axis_index("c")

  probe, _ = plsc.scan_count(jnp.zeros((L,), jnp.int32))
  bias = jnp.min(probe)
  one_m_bias = 1 - bias

  wlt = [(jnp.int32(w) < s).astype(jnp.int32) for w in range(NS)]

  pltpu.sync_copy(keys_hbm.at[pl.ds(s * CH, CH)], ck)

  def counting_pass(pass1, dst_ref):
    nbins = NB1 if pass1 else NB2
    nvb = nbins // L

    for b in range(nvb):
      cnt[pl.ds(b * L, L)] = jnp.zeros((L,), jnp.int32)

    @pl.loop(jnp.int32(0), jnp.int32(CH // L))
    def _local(i):
      sl = pl.ds(i * L, L)
      v = ck[sl]
      if pass1:
        idx = s * CH + i * L + lax.iota(jnp.int32, L)
        d = v & (NB1 - 1)
        ck[sl] = v * 32768 + idx
      else:
        d = lax.shift_right_logical(v, jnp.full((L,), 23, jnp.int32))
        ck[sl] = v & 32767
      run, last = plsc.scan_count(d)
      cur = plsc.load_gather(cnt, [d])
      dbuf[sl] = d
      plb[sl] = cur + run - bias
      plsc.addupdate_scatter(cnt, [d], run + one_m_bias, mask=last)

    pltpu.sync_copy(cnt.at[pl.ds(0, nbins)], gs_s.at[s, pl.ds(0, nbins)])
    plsc.subcore_barrier()
    pltpu.sync_copy(gs_s, gbuf)

    @pl.loop(jnp.int32(0), jnp.int32(nvb))
    def _sums(b):
      sl = pl.ds(b * L, L)
      tot = jnp.zeros((L,), jnp.int32)
      part = jnp.zeros((L,), jnp.int32)
      for w in range(NS):
        v = gbuf[w, sl]
        tot = tot + v
        part = part + v * wlt[w]
      dbuf[pl.ds(CH + b * L, L)] = tot
      cnt[sl] = part

    @pl.loop(jnp.int32(0), jnp.int32(nvb), init_carry=jnp.int32(0))
    def _scan(b, carry):
      sl = pl.ds(b * L, L)
      tot = dbuf[pl.ds(CH + b * L, L)]
      cnt[sl] = cnt[sl] + plsc.cumsum(tot) - tot + carry
      return carry + jnp.sum(tot, dtype=jnp.int32)

    descs = []
    for t in range(NBLK):
      for u in range(8):
        sl = pl.ds(t * 128 + u * L, L)
        d = dbuf[sl]
        posb[t, pl.ds(u * L, L)] = plsc.load_gather(cnt, [d]) + plb[sl]
      descs.append(pltpu.async_copy(
          ck.at[pl.ds(t * 128, 128)], dst_ref.at[posb.at[jnp.int32(t)]],
          sem_sc))
    for dsc in descs:
      dsc.wait()
    plsc.subcore_barrier()

  counting_pass(True, a_s)
  pltpu.sync_copy(a_s.at[pl.ds(s * CH, CH)], ck)
  counting_pass(False, ord_s)

  gbase = (s * NC + c) * GR
  for r in range(GR // 128):
    pltpu.sync_copy(ord_s.at[pl.ds(gbase + r * 128, 128)], idxb.at[jnp.int32(r)])

  bufs = (rowb0, rowb1)
  gsems = (sem_g0, sem_g1)
  nchunk = GR // 128
  gdescs = [None, None]
  wdescs = [None, None]
  gdescs[0] = pltpu.async_copy(values_hbm.at[idxb.at[jnp.int32(0)]],
                               bufs[0], gsems[0])
  for r in range(nchunk):
    b = r % 2
    if r + 1 < nchunk:
      if r >= 1:
        wdescs[(r + 1) % 2].wait()
      gdescs[(r + 1) % 2] = pltpu.async_copy(
          values_hbm.at[idxb.at[jnp.int32(r + 1)]], bufs[(r + 1) % 2],
          gsems[(r + 1) % 2])
    gdescs[b].wait()
    wdescs[b] = pltpu.async_copy(
        bufs[b], out_hbm.at[pl.ds(gbase + r * 128, 128)], sem_w)
  wdescs[(nchunk - 1) % 2].wait()
  wdescs[nchunk % 2].wait()


@jax.jit
def kernel(values, sample_ids):
  keys32 = sample_ids.astype(jnp.int32)
  mesh = plsc.VectorSubcoreMesh(
      core_axis_name="c", subcore_axis_name="s",
      num_cores=NC, num_subcores=NS)
  fn = pl.kernel(
      _body,
      out_type=jax.ShapeDtypeStruct((N, D), jnp.float32),
      mesh=mesh,
      scratch_types=[
          pltpu.VMEM((CH,), jnp.int32),
          pltpu.VMEM((CH + NB1,), jnp.int32),
          pltpu.VMEM((CH,), jnp.int32),
          pltpu.VMEM((NBLK, 128), jnp.int32),
          pltpu.VMEM((NB1,), jnp.int32),
          pltpu.VMEM((NS, NB1), jnp.int32),
          pltpu.VMEM((GR // 128, 128), jnp.int32),
          pltpu.VMEM((128, D), jnp.float32),
          pltpu.VMEM((128, D), jnp.float32),
          pltpu.VMEM_SHARED((NS, NB1), jnp.int32),
          pltpu.VMEM_SHARED((N,), jnp.int32),
          pltpu.VMEM_SHARED((N,), jnp.int32),
          pltpu.SemaphoreType.DMA,
          pltpu.SemaphoreType.DMA,
          pltpu.SemaphoreType.DMA,
          pltpu.SemaphoreType.DMA,
      ],
      compiler_params=pltpu.CompilerParams(needs_layout_passes=False),
      name="densify_sc",
  )
  return fn(values, keys32)
